# baseline (speedup 1.0000x reference)
import functools

import jax
import jax.numpy as jnp
from jax import lax
from jax.experimental import pallas as pl
from jax.experimental.pallas import tpu as pltpu
from jax.experimental.pallas import tpu_sc as plsc

N_NODES = 10000
N_EDGES = 160000
BATCH = 256

L = 16
NC = 2
NS = 16
NW = NC * NS
ROWS = BATCH // NW
PAIRS = ROWS // 2
CHUNK = 3200
N_CHUNKS = N_EDGES // CHUNK


def _energy_body(spins_hbm, edges_hbm, lin_hbm, out_hbm,
                 pk_v, st_a, st_b, lin_v, e_v, ob_v, esem0, esem1, ssem0, ssem1):
    wid = lax.axis_index("s") * NC + lax.axis_index("c")
    base = wid * (ROWS * N_NODES)

    esems = (esem0, esem1)
    ssems = (ssem0, ssem1)
    sts = (st_a, st_b)

    def fire_edges(c, slot):
        pltpu.async_copy(edges_hbm.at[pl.ds(c * 3 * CHUNK, 3 * CHUNK)],
                         e_v.at[slot], esems[slot])

    def drain_edges(slot):
        pltpu.make_async_copy(edges_hbm.at[pl.ds(0, 3 * CHUNK)],
                              e_v.at[slot], esems[slot]).wait()

    def fire_spins(p, slot):
        pltpu.async_copy(spins_hbm.at[pl.ds(base + 2 * p * N_NODES, 2 * N_NODES)],
                         sts[slot], ssems[slot])

    def drain_spins(slot):
        pltpu.make_async_copy(spins_hbm.at[pl.ds(0, 2 * N_NODES)],
                              sts[slot], ssems[slot]).wait()

    fire_spins(0, 0)
    fire_spins(1, 1)
    fire_edges(0, 0)
    fire_edges(1, 1)

    pltpu.sync_copy(lin_hbm, lin_v)

    zz = (jnp.zeros((L,), jnp.float32), jnp.zeros((L,), jnp.float32))
    accs = []
    for p in range(PAIRS):
        sslot = p % 2
        drain_spins(sslot)

        @plsc.parallel_loop(0, N_NODES // L, unroll=2, carry=zz)
        def lacc(v, lacc):
            a = sts[sslot][pl.ds(v * L, L)]
            b = sts[sslot][pl.ds(N_NODES + v * L, L)]
            pk_v[pl.ds(p * N_NODES + v * L, L)] = plsc.bitcast(
                plsc.pack(a, b, format=plsc.PackFormat.INTERLEAVED), jnp.int32)
            lv = lin_v[pl.ds(v * L, L)]
            return (lacc[0] + a * lv, lacc[1] + b * lv)

        accs += [lacc[0], lacc[1]]
        if p + 2 < PAIRS:
            fire_spins(p + 2, sslot)
    accs = tuple(accs)

    def chunk_pair(g, accs):
        for slot in range(2):
            c = g * 2 + slot
            drain_edges(slot)

            @plsc.parallel_loop(0, CHUNK // L, unroll=4, carry=accs)
            def accs(v, accs):
                iv = e_v[slot, pl.ds(v * L, L)]
                jv = e_v[slot, pl.ds(CHUNK + v * L, L)]
                qv = plsc.bitcast(e_v[slot, pl.ds(2 * CHUNK + v * L, L)], jnp.float32)
                new = list(accs)
                for p in range(PAIRS):
                    ga = plsc.load_gather(pk_v, [iv + p * N_NODES])
                    gb = plsc.load_gather(pk_v, [jv + p * N_NODES])
                    prod = plsc.bitcast(ga, jnp.bfloat16) * plsc.bitcast(gb, jnp.bfloat16)
                    lo, hi = plsc.unpack(prod, format=plsc.PackFormat.INTERLEAVED)
                    new[2 * p] = new[2 * p] + qv * lo
                    new[2 * p + 1] = new[2 * p + 1] + qv * hi
                return tuple(new)

            @pl.when(c + 2 < N_CHUNKS)
            def _():
                fire_edges(c + 2, slot)
        return accs

    accs = lax.fori_loop(0, N_CHUNKS // 2, chunk_pair, accs)

    lane = lax.iota(jnp.int32, L)
    ob = jnp.zeros((L,), jnp.float32)
    for r in range(ROWS):
        ob = jnp.where(lane == r, jnp.sum(accs[r]), ob)
    ob_v[...] = ob
    pltpu.sync_copy(ob_v, out_hbm.at[wid])


_energy_kernel = functools.partial(
    pl.kernel,
    out_type=jax.ShapeDtypeStruct((NW, L), jnp.float32),
    mesh=plsc.VectorSubcoreMesh(core_axis_name="c", subcore_axis_name="s"),
    compiler_params=pltpu.CompilerParams(needs_layout_passes=False),
    scratch_types=[
        pltpu.VMEM((PAIRS * N_NODES,), jnp.int32),
        pltpu.VMEM((2 * N_NODES,), jnp.float32),
        pltpu.VMEM((2 * N_NODES,), jnp.float32),
        pltpu.VMEM((N_NODES,), jnp.float32),
        pltpu.VMEM((2, 3 * CHUNK), jnp.int32),
        pltpu.VMEM((L,), jnp.float32),
        pltpu.SemaphoreType.DMA,
        pltpu.SemaphoreType.DMA,
        pltpu.SemaphoreType.DMA,
        pltpu.SemaphoreType.DMA,
    ],
)(_energy_body)


def kernel(spins, edge_idx_i, edge_idx_j, linear, quadratic):
    edges = jnp.concatenate(
        [edge_idx_i.astype(jnp.int32).reshape(N_CHUNKS, 1, CHUNK),
         edge_idx_j.astype(jnp.int32).reshape(N_CHUNKS, 1, CHUNK),
         jax.lax.bitcast_convert_type(quadratic, jnp.int32).reshape(N_CHUNKS, 1, CHUNK)],
        axis=1)
    out2d = _energy_kernel(spins.reshape(-1), edges.reshape(-1), linear)
    return out2d[:, :ROWS].reshape(BATCH)


# speedup vs baseline: 1.0583x; 1.0583x over previous
import functools

import jax
import jax.numpy as jnp
from jax import lax
from jax.experimental import pallas as pl
from jax.experimental.pallas import tpu as pltpu
from jax.experimental.pallas import tpu_sc as plsc

N_NODES = 10000
N_EDGES = 160000
BATCH = 256

L = 16
NC = 2
NS = 16
NW = NC * NS
ROWS = BATCH // NW
PAIRS = ROWS // 2
CHUNK = 3200
N_CHUNKS = N_EDGES // CHUNK


def _energy_body(spins_hbm, ii_hbm, jj_hbm, lin_hbm, q_hbm, out_hbm,
                 pk_v, st_a, st_b, lin_v, iv_v, jv_v, qv_v, ob_v,
                 esem0, esem1, ssem0, ssem1):
    wid = lax.axis_index("s") * NC + lax.axis_index("c")
    base = wid * (ROWS * N_NODES)

    esems = (esem0, esem1)
    ssems = (ssem0, ssem1)
    sts = (st_a, st_b)

    def fire_edges(c, slot):
        off = c * CHUNK
        pltpu.async_copy(ii_hbm.at[pl.ds(off, CHUNK)], iv_v.at[slot], esems[slot])
        pltpu.async_copy(jj_hbm.at[pl.ds(off, CHUNK)], jv_v.at[slot], esems[slot])
        pltpu.async_copy(q_hbm.at[pl.ds(off, CHUNK)], qv_v.at[slot], esems[slot])

    def drain_edges(slot):
        pltpu.make_async_copy(ii_hbm.at[pl.ds(0, CHUNK)], iv_v.at[slot], esems[slot]).wait()
        pltpu.make_async_copy(jj_hbm.at[pl.ds(0, CHUNK)], jv_v.at[slot], esems[slot]).wait()
        pltpu.make_async_copy(q_hbm.at[pl.ds(0, CHUNK)], qv_v.at[slot], esems[slot]).wait()

    def fire_spins(p, slot):
        pltpu.async_copy(spins_hbm.at[pl.ds(base + 2 * p * N_NODES, 2 * N_NODES)],
                         sts[slot], ssems[slot])

    def drain_spins(slot):
        pltpu.make_async_copy(spins_hbm.at[pl.ds(0, 2 * N_NODES)],
                              sts[slot], ssems[slot]).wait()

    fire_spins(0, 0)
    fire_spins(1, 1)
    fire_edges(0, 0)
    fire_edges(1, 1)

    pltpu.sync_copy(lin_hbm, lin_v)

    zz = (jnp.zeros((L,), jnp.float32), jnp.zeros((L,), jnp.float32))
    accs = []
    for p in range(PAIRS):
        sslot = p % 2
        drain_spins(sslot)

        @plsc.parallel_loop(0, N_NODES // L, unroll=2, carry=zz)
        def lacc(v, lacc):
            a = sts[sslot][pl.ds(v * L, L)]
            b = sts[sslot][pl.ds(N_NODES + v * L, L)]
            pk_v[pl.ds(p * N_NODES + v * L, L)] = plsc.bitcast(
                plsc.pack(a, b, format=plsc.PackFormat.INTERLEAVED), jnp.int32)
            lv = lin_v[pl.ds(v * L, L)]
            return (lacc[0] + a * lv, lacc[1] + b * lv)

        accs += [lacc[0], lacc[1]]
        if p + 2 < PAIRS:
            fire_spins(p + 2, sslot)
    accs = tuple(accs)

    def chunk_pair(g, accs):
        for slot in range(2):
            c = g * 2 + slot
            drain_edges(slot)

            @plsc.parallel_loop(0, CHUNK // L, unroll=4, carry=accs)
            def accs(v, accs):
                iv = iv_v[slot, pl.ds(v * L, L)]
                jv = jv_v[slot, pl.ds(v * L, L)]
                qv = qv_v[slot, pl.ds(v * L, L)]
                new = list(accs)
                for p in range(PAIRS):
                    ga = plsc.load_gather(pk_v, [iv + p * N_NODES])
                    gb = plsc.load_gather(pk_v, [jv + p * N_NODES])
                    prod = plsc.bitcast(ga, jnp.bfloat16) * plsc.bitcast(gb, jnp.bfloat16)
                    lo, hi = plsc.unpack(prod, format=plsc.PackFormat.INTERLEAVED)
                    new[2 * p] = new[2 * p] + qv * lo
                    new[2 * p + 1] = new[2 * p + 1] + qv * hi
                return tuple(new)

            @pl.when(c + 2 < N_CHUNKS)
            def _():
                fire_edges(c + 2, slot)
        return accs

    accs = lax.fori_loop(0, N_CHUNKS // 2, chunk_pair, accs)

    lane = lax.iota(jnp.int32, L)
    ob = jnp.zeros((L,), jnp.float32)
    for r in range(ROWS):
        ob = jnp.where(lane == r, jnp.sum(accs[r]), ob)
    ob_v[...] = ob
    pltpu.sync_copy(ob_v, out_hbm.at[wid])


_energy_kernel = functools.partial(
    pl.kernel,
    out_type=jax.ShapeDtypeStruct((NW, L), jnp.float32),
    mesh=plsc.VectorSubcoreMesh(core_axis_name="c", subcore_axis_name="s"),
    compiler_params=pltpu.CompilerParams(needs_layout_passes=False),
    scratch_types=[
        pltpu.VMEM((PAIRS * N_NODES,), jnp.int32),
        pltpu.VMEM((2 * N_NODES,), jnp.float32),
        pltpu.VMEM((2 * N_NODES,), jnp.float32),
        pltpu.VMEM((N_NODES,), jnp.float32),
        pltpu.VMEM((2, CHUNK), jnp.int32),
        pltpu.VMEM((2, CHUNK), jnp.int32),
        pltpu.VMEM((2, CHUNK), jnp.float32),
        pltpu.VMEM((L,), jnp.float32),
        pltpu.SemaphoreType.DMA,
        pltpu.SemaphoreType.DMA,
        pltpu.SemaphoreType.DMA,
        pltpu.SemaphoreType.DMA,
    ],
)(_energy_body)


def kernel(spins, edge_idx_i, edge_idx_j, linear, quadratic):
    out2d = _energy_kernel(spins.reshape(-1), edge_idx_i.astype(jnp.int32),
                           edge_idx_j.astype(jnp.int32), linear, quadratic)
    return out2d[:, :ROWS].reshape(BATCH)


# speedup vs baseline: 1.0602x; 1.0018x over previous
import functools

import jax
import jax.numpy as jnp
from jax import lax
from jax.experimental import pallas as pl
from jax.experimental.pallas import tpu as pltpu
from jax.experimental.pallas import tpu_sc as plsc

N_NODES = 10000
N_EDGES = 160000
BATCH = 256

L = 16
NC = 2
NS = 16
NW = NC * NS
ROWS = BATCH // NW
PAIRS = ROWS // 2
CHUNK = 3200
N_CHUNKS = N_EDGES // CHUNK


def _energy_body(spins_hbm, ii_hbm, jj_hbm, lin_hbm, q_hbm, out_hbm,
                 pk0_v, pk1_v, pk2_v, pk3_v, st_a, st_b, lin_v,
                 iv_v, jv_v, qv_v, ob_v, esem0, esem1, ssem0, ssem1):
    wid = lax.axis_index("s") * NC + lax.axis_index("c")
    base = wid * (ROWS * N_NODES)

    esems = (esem0, esem1)
    ssems = (ssem0, ssem1)
    sts = (st_a, st_b)
    pks = (pk0_v, pk1_v, pk2_v, pk3_v)

    def fire_edges(c, slot):
        off = c * CHUNK
        pltpu.async_copy(ii_hbm.at[pl.ds(off, CHUNK)], iv_v.at[slot], esems[slot])
        pltpu.async_copy(jj_hbm.at[pl.ds(off, CHUNK)], jv_v.at[slot], esems[slot])
        pltpu.async_copy(q_hbm.at[pl.ds(off, CHUNK)], qv_v.at[slot], esems[slot])

    def drain_edges(slot):
        pltpu.make_async_copy(ii_hbm.at[pl.ds(0, CHUNK)], iv_v.at[slot], esems[slot]).wait()
        pltpu.make_async_copy(jj_hbm.at[pl.ds(0, CHUNK)], jv_v.at[slot], esems[slot]).wait()
        pltpu.make_async_copy(q_hbm.at[pl.ds(0, CHUNK)], qv_v.at[slot], esems[slot]).wait()

    def fire_spins(p, slot):
        pltpu.async_copy(spins_hbm.at[pl.ds(base + 2 * p * N_NODES, 2 * N_NODES)],
                         sts[slot], ssems[slot])

    def drain_spins(slot):
        pltpu.make_async_copy(spins_hbm.at[pl.ds(0, 2 * N_NODES)],
                              sts[slot], ssems[slot]).wait()

    fire_spins(0, 0)
    fire_spins(1, 1)
    fire_edges(0, 0)
    fire_edges(1, 1)

    pltpu.sync_copy(lin_hbm, lin_v)

    zz = (jnp.zeros((L,), jnp.float32), jnp.zeros((L,), jnp.float32))
    accs = []
    for p in range(PAIRS):
        sslot = p % 2
        drain_spins(sslot)

        @plsc.parallel_loop(0, N_NODES // L, unroll=2, carry=zz)
        def lacc(v, lacc):
            a = sts[sslot][pl.ds(v * L, L)]
            b = sts[sslot][pl.ds(N_NODES + v * L, L)]
            pks[p][pl.ds(v * L, L)] = plsc.bitcast(
                plsc.pack(a, b, format=plsc.PackFormat.INTERLEAVED), jnp.int32)
            lv = lin_v[pl.ds(v * L, L)]
            return (lacc[0] + a * lv, lacc[1] + b * lv)

        accs += [lacc[0], lacc[1]]
        if p + 2 < PAIRS:
            fire_spins(p + 2, sslot)
    accs = tuple(accs)

    def chunk_pair(g, accs):
        for slot in range(2):
            c = g * 2 + slot
            drain_edges(slot)

            @plsc.parallel_loop(0, CHUNK // L, unroll=4, carry=accs)
            def accs(v, accs):
                iv = iv_v[slot, pl.ds(v * L, L)]
                jv = jv_v[slot, pl.ds(v * L, L)]
                qv = qv_v[slot, pl.ds(v * L, L)]
                new = list(accs)
                for p in range(PAIRS):
                    ga = plsc.load_gather(pks[p], [iv])
                    gb = plsc.load_gather(pks[p], [jv])
                    prod = plsc.bitcast(ga, jnp.bfloat16) * plsc.bitcast(gb, jnp.bfloat16)
                    lo, hi = plsc.unpack(prod, format=plsc.PackFormat.INTERLEAVED)
                    new[2 * p] = new[2 * p] + qv * lo
                    new[2 * p + 1] = new[2 * p + 1] + qv * hi
                return tuple(new)

            @pl.when(c + 2 < N_CHUNKS)
            def _():
                fire_edges(c + 2, slot)
        return accs

    accs = lax.fori_loop(0, N_CHUNKS // 2, chunk_pair, accs)

    lane = lax.iota(jnp.int32, L)
    ob = jnp.zeros((L,), jnp.float32)
    for r in range(ROWS):
        ob = jnp.where(lane == r, jnp.sum(accs[r]), ob)
    ob_v[...] = ob
    pltpu.sync_copy(ob_v, out_hbm.at[wid])


_energy_kernel = functools.partial(
    pl.kernel,
    out_type=jax.ShapeDtypeStruct((NW, L), jnp.float32),
    mesh=plsc.VectorSubcoreMesh(core_axis_name="c", subcore_axis_name="s"),
    compiler_params=pltpu.CompilerParams(needs_layout_passes=False),
    scratch_types=[
        pltpu.VMEM((N_NODES,), jnp.int32),
        pltpu.VMEM((N_NODES,), jnp.int32),
        pltpu.VMEM((N_NODES,), jnp.int32),
        pltpu.VMEM((N_NODES,), jnp.int32),
        pltpu.VMEM((2 * N_NODES,), jnp.float32),
        pltpu.VMEM((2 * N_NODES,), jnp.float32),
        pltpu.VMEM((N_NODES,), jnp.float32),
        pltpu.VMEM((2, CHUNK), jnp.int32),
        pltpu.VMEM((2, CHUNK), jnp.int32),
        pltpu.VMEM((2, CHUNK), jnp.float32),
        pltpu.VMEM((L,), jnp.float32),
        pltpu.SemaphoreType.DMA,
        pltpu.SemaphoreType.DMA,
        pltpu.SemaphoreType.DMA,
        pltpu.SemaphoreType.DMA,
    ],
)(_energy_body)


def kernel(spins, edge_idx_i, edge_idx_j, linear, quadratic):
    out2d = _energy_kernel(spins.reshape(-1), edge_idx_i.astype(jnp.int32),
                           edge_idx_j.astype(jnp.int32), linear, quadratic)
    return out2d[:, :ROWS].reshape(BATCH)


# speedup vs baseline: 1.1469x; 1.0818x over previous
import functools

import jax
import jax.numpy as jnp
from jax import lax
from jax.experimental import pallas as pl
from jax.experimental.pallas import tpu as pltpu
from jax.experimental.pallas import tpu_sc as plsc

N_NODES = 10000
N_EDGES = 160000
BATCH = 256

L = 16
NC = 2
NS = 16
NW = NC * NS
ROWS = BATCH // NW
PAIRS = ROWS // 2
CHUNK = 3200
N_CHUNKS = N_EDGES // CHUNK


def _energy_body(spins_hbm, ij_hbm, lin_hbm, q_hbm, out_hbm,
                 pk0_v, pk1_v, pk2_v, pk3_v, st_a, st_b, lin_v,
                 ij_v, qv_v, ob_v, esem0, esem1, ssem0, ssem1):
    wid = lax.axis_index("s") * NC + lax.axis_index("c")
    base = wid * (ROWS * N_NODES)

    esems = (esem0, esem1)
    ssems = (ssem0, ssem1)
    sts = (st_a, st_b)
    pks = (pk0_v, pk1_v, pk2_v, pk3_v)

    def fire_edges(c, slot):
        off = c * CHUNK
        pltpu.async_copy(ij_hbm.at[pl.ds(off, CHUNK)], ij_v.at[slot], esems[slot])
        pltpu.async_copy(q_hbm.at[pl.ds(off, CHUNK)], qv_v.at[slot], esems[slot])

    def drain_edges(slot):
        pltpu.make_async_copy(ij_hbm.at[pl.ds(0, CHUNK)], ij_v.at[slot], esems[slot]).wait()
        pltpu.make_async_copy(q_hbm.at[pl.ds(0, CHUNK)], qv_v.at[slot], esems[slot]).wait()

    def fire_spins(p, slot):
        pltpu.async_copy(spins_hbm.at[pl.ds(base + 2 * p * N_NODES, 2 * N_NODES)],
                         sts[slot], ssems[slot])

    def drain_spins(slot):
        pltpu.make_async_copy(spins_hbm.at[pl.ds(0, 2 * N_NODES)],
                              sts[slot], ssems[slot]).wait()

    fire_spins(0, 0)
    fire_spins(1, 1)
    fire_edges(0, 0)
    fire_edges(1, 1)

    pltpu.sync_copy(lin_hbm, lin_v)

    zz = (jnp.zeros((L,), jnp.float32), jnp.zeros((L,), jnp.float32))
    accs = []
    for p in range(PAIRS):
        sslot = p % 2
        drain_spins(sslot)

        @plsc.parallel_loop(0, N_NODES // L, unroll=2, carry=zz)
        def lacc(v, lacc):
            a = sts[sslot][pl.ds(v * L, L)]
            b = sts[sslot][pl.ds(N_NODES + v * L, L)]
            pks[p][pl.ds(v * L, L)] = plsc.bitcast(
                plsc.pack(a, b, format=plsc.PackFormat.INTERLEAVED), jnp.int32)
            lv = lin_v[pl.ds(v * L, L)]
            return (lacc[0] + a * lv, lacc[1] + b * lv)

        accs += [lacc[0], lacc[1]]
        if p + 2 < PAIRS:
            fire_spins(p + 2, sslot)
    accs = tuple(accs)

    def chunk_pair(g, accs):
        for slot in range(2):
            c = g * 2 + slot
            drain_edges(slot)

            @plsc.parallel_loop(0, CHUNK // L, unroll=4, carry=accs)
            def accs(v, accs):
                ij = ij_v[slot, pl.ds(v * L, L)]
                iv = ij & 0xFFFF
                jv = lax.shift_right_logical(ij, 16)
                qv = qv_v[slot, pl.ds(v * L, L)]
                new = list(accs)
                for p in range(PAIRS):
                    ga = plsc.load_gather(pks[p], [iv])
                    gb = plsc.load_gather(pks[p], [jv])
                    prod = plsc.bitcast(ga, jnp.bfloat16) * plsc.bitcast(gb, jnp.bfloat16)
                    lo, hi = plsc.unpack(prod, format=plsc.PackFormat.INTERLEAVED)
                    new[2 * p] = new[2 * p] + qv * lo
                    new[2 * p + 1] = new[2 * p + 1] + qv * hi
                return tuple(new)

            @pl.when(c + 2 < N_CHUNKS)
            def _():
                fire_edges(c + 2, slot)
        return accs

    accs = lax.fori_loop(0, N_CHUNKS // 2, chunk_pair, accs)

    lane = lax.iota(jnp.int32, L)
    ob = jnp.zeros((L,), jnp.float32)
    for r in range(ROWS):
        ob = jnp.where(lane == r, jnp.sum(accs[r]), ob)
    ob_v[...] = ob
    pltpu.sync_copy(ob_v, out_hbm.at[wid])


_energy_kernel = functools.partial(
    pl.kernel,
    out_type=jax.ShapeDtypeStruct((NW, L), jnp.float32),
    mesh=plsc.VectorSubcoreMesh(core_axis_name="c", subcore_axis_name="s"),
    compiler_params=pltpu.CompilerParams(needs_layout_passes=False),
    scratch_types=[
        pltpu.VMEM((N_NODES,), jnp.int32),
        pltpu.VMEM((N_NODES,), jnp.int32),
        pltpu.VMEM((N_NODES,), jnp.int32),
        pltpu.VMEM((N_NODES,), jnp.int32),
        pltpu.VMEM((2 * N_NODES,), jnp.float32),
        pltpu.VMEM((2 * N_NODES,), jnp.float32),
        pltpu.VMEM((N_NODES,), jnp.float32),
        pltpu.VMEM((2, CHUNK), jnp.int32),
        pltpu.VMEM((2, CHUNK), jnp.float32),
        pltpu.VMEM((L,), jnp.float32),
        pltpu.SemaphoreType.DMA,
        pltpu.SemaphoreType.DMA,
        pltpu.SemaphoreType.DMA,
        pltpu.SemaphoreType.DMA,
    ],
)(_energy_body)


def kernel(spins, edge_idx_i, edge_idx_j, linear, quadratic):
    ij = edge_idx_i.astype(jnp.int32) | (edge_idx_j.astype(jnp.int32) << 16)
    out2d = _energy_kernel(spins.reshape(-1), ij, linear, quadratic)
    return out2d[:, :ROWS].reshape(BATCH)
